# baseline (device time: 16324 ns/iter reference)
import jax
import jax.numpy as jnp
from jax import lax
from jax.experimental import pallas as pl
from jax.experimental.pallas import tpu as pltpu

N_DEV = 4
B, SQ, SKV, DH = 2, 128, 128, 64
H_LOC = 4
D_MODEL = 512
D_LOC = H_LOC * DH


def kernel(x, Wq, K_ext, V_ext, Wo):
    my = lax.axis_index("i")
    xf = x.reshape(B * SQ, D_MODEL)
    K2 = lax.dynamic_slice_in_dim(K_ext, my * H_LOC, H_LOC, axis=2)
    K2 = K2.reshape(B * SKV, D_LOC)
    V2 = lax.dynamic_slice_in_dim(V_ext, my * H_LOC, H_LOC, axis=2)
    V2 = V2.reshape(B * SKV, D_LOC)

    def body(x_ref, wq_ref, k_ref, v_ref, wo_ref, out_ref,
             ctx_ref, send_ref, recv_ref, send_sems, recv_sems):
        my_pos = lax.axis_index("i")
        p1 = my_pos ^ 1
        p2 = (N_DEV - 1) - my_pos

        barrier_sem = pltpu.get_barrier_semaphore()
        for nbr in (p1, p2):
            pl.semaphore_signal(
                barrier_sem, inc=1,
                device_id=(nbr,), device_id_type=pl.DeviceIdType.MESH,
            )
        pl.semaphore_wait(barrier_sem, 2)

        xb = x_ref[:].astype(jnp.bfloat16)
        wq = wq_ref[:].astype(jnp.bfloat16)
        q = lax.dot(xb, wq, preferred_element_type=jnp.float32)
        qb = (q * 0.125).astype(jnp.bfloat16)
        kb = k_ref[:].astype(jnp.bfloat16)
        vb = v_ref[:].astype(jnp.bfloat16)
        wo = wo_ref[:].astype(jnp.bfloat16)

        def exchange(round_idx, b, partner):
            return pltpu.make_async_remote_copy(
                src_ref=send_ref.at[round_idx, b],
                dst_ref=recv_ref.at[round_idx, b],
                send_sem=send_sems.at[round_idx, b],
                recv_sem=recv_sems.at[round_idx, b],
                device_id=(partner,),
                device_id_type=pl.DeviceIdType.MESH,
            )

        r1 = []
        for b in range(B):
            rows = slice(b * SQ, (b + 1) * SQ)
            for h in range(H_LOC):
                cols = slice(h * DH, (h + 1) * DH)
                q_bh = qb[rows, cols]
                k_bh = kb[rows, cols]
                v_bh = vb[rows, cols]
                s = lax.dot_general(
                    q_bh, k_bh, (((1,), (1,)), ((), ())),
                    preferred_element_type=jnp.float32,
                )
                m = jnp.max(s, axis=1, keepdims=True)
                e = jnp.exp(s - m)
                w = e / jnp.sum(e, axis=1, keepdims=True)
                ctx_bh = lax.dot(
                    w.astype(jnp.bfloat16), v_bh,
                    preferred_element_type=jnp.float32,
                )
                ctx_ref[rows, cols] = ctx_bh.astype(jnp.bfloat16)

            partial_b = lax.dot(
                ctx_ref[rows, :], wo, preferred_element_type=jnp.float32
            )
            out_ref[rows, :] = partial_b
            send_ref[0, b] = partial_b.astype(jnp.bfloat16)
            rdma = exchange(0, b, p1)
            rdma.start()
            r1.append(rdma)

        r2 = []
        for b in range(B):
            rows = slice(b * SQ, (b + 1) * SQ)
            r1[b].wait_recv()
            out_ref[rows, :] += recv_ref[0, b].astype(jnp.float32)
            send_ref[1, b] = out_ref[rows, :].astype(jnp.bfloat16)
            rdma = exchange(1, b, p2)
            rdma.start()
            r2.append(rdma)

        for b in range(B):
            rows = slice(b * SQ, (b + 1) * SQ)
            r2[b].wait_recv()
            out_ref[rows, :] += recv_ref[1, b].astype(jnp.float32)

        for rdma in r1 + r2:
            rdma.wait_send()

    out = pl.pallas_call(
        body,
        out_shape=jax.ShapeDtypeStruct((B * SQ, D_MODEL), jnp.float32),
        in_specs=[pl.BlockSpec(memory_space=pltpu.VMEM)] * 5,
        out_specs=pl.BlockSpec(memory_space=pltpu.VMEM),
        scratch_shapes=[
            pltpu.VMEM((B * SQ, D_LOC), jnp.bfloat16),
            pltpu.VMEM((2, B, SQ, D_MODEL), jnp.bfloat16),
            pltpu.VMEM((2, B, SQ, D_MODEL), jnp.bfloat16),
            pltpu.SemaphoreType.DMA((2, B)),
            pltpu.SemaphoreType.DMA((2, B)),
        ],
        compiler_params=pltpu.CompilerParams(collective_id=0),
    )(xf, Wq, K2, V2, Wo)
    return out.reshape(B, SQ, D_MODEL)


# device time: 15451 ns/iter; 1.0565x vs baseline; 1.0565x over previous
import jax
import jax.numpy as jnp
from jax import lax
from jax.experimental import pallas as pl
from jax.experimental.pallas import tpu as pltpu

N_DEV = 4
B, SQ, SKV, DH = 2, 128, 128, 64
H_LOC = 4
D_MODEL = 512
D_LOC = H_LOC * DH


def kernel(x, Wq, K_ext, V_ext, Wo):
    my = lax.axis_index("i")
    xf = x.reshape(B * SQ, D_MODEL)
    K2 = lax.dynamic_slice_in_dim(K_ext, my * H_LOC, H_LOC, axis=2)
    K2 = K2.reshape(B * SKV, D_LOC)
    V2 = lax.dynamic_slice_in_dim(V_ext, my * H_LOC, H_LOC, axis=2)
    V2 = V2.reshape(B * SKV, D_LOC)

    def body(x_ref, wq_ref, k_ref, v_ref, wo_ref, out_ref,
             ctx_ref, send_ref, recv_ref, send_sems, recv_sems):
        my_pos = lax.axis_index("i")
        p1 = my_pos ^ 1
        p2 = (N_DEV - 1) - my_pos

        barrier_sem = pltpu.get_barrier_semaphore()
        for nbr in (p1, p2):
            pl.semaphore_signal(
                barrier_sem, inc=1,
                device_id=(nbr,), device_id_type=pl.DeviceIdType.MESH,
            )
        pl.semaphore_wait(barrier_sem, 2)

        xb = x_ref[:].astype(jnp.bfloat16)
        wq = wq_ref[:].astype(jnp.bfloat16)
        q = lax.dot(xb, wq, preferred_element_type=jnp.float32)
        qb = (q * 0.125).astype(jnp.bfloat16)
        kb = k_ref[:].astype(jnp.bfloat16)
        vb = v_ref[:].astype(jnp.bfloat16)
        wo = wo_ref[:].astype(jnp.bfloat16)

        def exchange(round_idx, b, partner):
            return pltpu.make_async_remote_copy(
                src_ref=send_ref.at[round_idx, b],
                dst_ref=recv_ref.at[round_idx, b],
                send_sem=send_sems.at[round_idx, b],
                recv_sem=recv_sems.at[round_idx, b],
                device_id=(partner,),
                device_id_type=pl.DeviceIdType.MESH,
            )

        r1 = []
        for b in range(B):
            rows = slice(b * SQ, (b + 1) * SQ)
            for h in range(H_LOC):
                cols = slice(h * DH, (h + 1) * DH)
                q_bh = qb[rows, cols]
                k_bh = kb[rows, cols]
                v_bh = vb[rows, cols]
                s = lax.dot_general(
                    q_bh, k_bh, (((1,), (1,)), ((), ())),
                    preferred_element_type=jnp.float32,
                )
                e = jnp.exp(s)
                recip = 1.0 / jnp.sum(e, axis=1, keepdims=True)
                ctx_bh = lax.dot(
                    e.astype(jnp.bfloat16), v_bh,
                    preferred_element_type=jnp.float32,
                ) * recip
                ctx_ref[rows, cols] = ctx_bh.astype(jnp.bfloat16)

            partial_b = lax.dot(
                ctx_ref[rows, :], wo, preferred_element_type=jnp.float32
            )
            out_ref[rows, :] = partial_b
            send_ref[0, b] = partial_b.astype(jnp.bfloat16)
            rdma = exchange(0, b, p1)
            rdma.start()
            r1.append(rdma)

        r2 = []
        for b in range(B):
            rows = slice(b * SQ, (b + 1) * SQ)
            r1[b].wait_recv()
            out_ref[rows, :] += recv_ref[0, b].astype(jnp.float32)
            send_ref[1, b] = out_ref[rows, :].astype(jnp.bfloat16)
            rdma = exchange(1, b, p2)
            rdma.start()
            r2.append(rdma)

        for b in range(B):
            rows = slice(b * SQ, (b + 1) * SQ)
            r2[b].wait_recv()
            out_ref[rows, :] += recv_ref[1, b].astype(jnp.float32)

        for rdma in r1 + r2:
            rdma.wait_send()

    out = pl.pallas_call(
        body,
        out_shape=jax.ShapeDtypeStruct((B * SQ, D_MODEL), jnp.float32),
        in_specs=[pl.BlockSpec(memory_space=pltpu.VMEM)] * 5,
        out_specs=pl.BlockSpec(memory_space=pltpu.VMEM),
        scratch_shapes=[
            pltpu.VMEM((B * SQ, D_LOC), jnp.bfloat16),
            pltpu.VMEM((2, B, SQ, D_MODEL), jnp.bfloat16),
            pltpu.VMEM((2, B, SQ, D_MODEL), jnp.bfloat16),
            pltpu.SemaphoreType.DMA((2, B)),
            pltpu.SemaphoreType.DMA((2, B)),
        ],
        compiler_params=pltpu.CompilerParams(collective_id=0),
    )(xf, Wq, K2, V2, Wo)
    return out.reshape(B, SQ, D_MODEL)


# device time: 15365 ns/iter; 1.0624x vs baseline; 1.0056x over previous
import jax
import jax.numpy as jnp
from jax import lax
from jax.experimental import pallas as pl
from jax.experimental.pallas import tpu as pltpu

N_DEV = 4
B, SQ, SKV, DH = 2, 128, 128, 64
H_LOC = 4
D_MODEL = 512
D_LOC = H_LOC * DH


def kernel(x, Wq, K_ext, V_ext, Wo):
    def body(x_ref, wq_ref, k_ref, v_ref, wo_ref, out_ref,
             send_ref, recv_ref, send_sems, recv_sems):
        my_pos = lax.axis_index("i")
        p1 = my_pos ^ 1
        p2 = (N_DEV - 1) - my_pos

        barrier_sem = pltpu.get_barrier_semaphore()
        for nbr in (p1, p2):
            pl.semaphore_signal(
                barrier_sem, inc=1,
                device_id=(nbr,), device_id_type=pl.DeviceIdType.MESH,
            )
        pl.semaphore_wait(barrier_sem, 2)

        wq = wq_ref[:].astype(jnp.bfloat16)
        wo = wo_ref[:].astype(jnp.bfloat16)

        def exchange(round_idx, b, partner):
            return pltpu.make_async_remote_copy(
                src_ref=send_ref.at[round_idx, b],
                dst_ref=recv_ref.at[round_idx, b],
                send_sem=send_sems.at[round_idx, b],
                recv_sem=recv_sems.at[round_idx, b],
                device_id=(partner,),
                device_id_type=pl.DeviceIdType.MESH,
            )

        r1 = []
        for b in range(B):
            xb = x_ref[b].astype(jnp.bfloat16)
            q_b = lax.dot(xb, wq, preferred_element_type=jnp.float32)
            q_b = (q_b * 0.125).astype(jnp.bfloat16)
            ctx = []
            for h in range(H_LOC):
                q_bh = q_b[:, h * DH:(h + 1) * DH]
                k_bh = k_ref[b, :, h * DH:(h + 1) * DH].astype(jnp.bfloat16)
                v_bh = v_ref[b, :, h * DH:(h + 1) * DH].astype(jnp.bfloat16)
                s = lax.dot_general(
                    q_bh, k_bh, (((1,), (1,)), ((), ())),
                    preferred_element_type=jnp.float32,
                )
                e = jnp.exp(s)
                recip = 1.0 / jnp.sum(e, axis=1, keepdims=True)
                ctx_bh = lax.dot(
                    e.astype(jnp.bfloat16), v_bh,
                    preferred_element_type=jnp.float32,
                ) * recip
                ctx.append(ctx_bh.astype(jnp.bfloat16))

            ctx_b = jnp.concatenate(ctx, axis=1)
            partial_b = lax.dot(
                ctx_b, wo, preferred_element_type=jnp.float32
            )
            out_ref[b] = partial_b
            send_ref[0, b] = partial_b.astype(jnp.bfloat16)
            rdma = exchange(0, b, p1)
            rdma.start()
            r1.append(rdma)

        r2 = []
        for b in range(B):
            r1[b].wait_recv()
            out_ref[b] += recv_ref[0, b].astype(jnp.float32)
            send_ref[1, b] = out_ref[b].astype(jnp.bfloat16)
            rdma = exchange(1, b, p2)
            rdma.start()
            r2.append(rdma)

        for b in range(B):
            r2[b].wait_recv()
            out_ref[b] += recv_ref[1, b].astype(jnp.float32)

        for rdma in r1 + r2:
            rdma.wait_send()

    def kv_map(i):
        return (0, 0, lax.axis_index("i"))

    return pl.pallas_call(
        body,
        grid=(1,),
        out_shape=jax.ShapeDtypeStruct((B, SQ, D_MODEL), jnp.float32),
        in_specs=[
            pl.BlockSpec((B, SQ, D_MODEL), lambda i: (0, 0, 0)),
            pl.BlockSpec((D_MODEL, D_LOC), lambda i: (0, 0)),
            pl.BlockSpec((B, SKV, D_LOC), kv_map),
            pl.BlockSpec((B, SKV, D_LOC), kv_map),
            pl.BlockSpec((D_LOC, D_MODEL), lambda i: (0, 0)),
        ],
        out_specs=pl.BlockSpec((B, SQ, D_MODEL), lambda i: (0, 0, 0)),
        scratch_shapes=[
            pltpu.VMEM((2, B, SQ, D_MODEL), jnp.bfloat16),
            pltpu.VMEM((2, B, SQ, D_MODEL), jnp.bfloat16),
            pltpu.SemaphoreType.DMA((2, B)),
            pltpu.SemaphoreType.DMA((2, B)),
        ],
        compiler_params=pltpu.CompilerParams(collective_id=0),
    )(
        x,
        Wq,
        K_ext.reshape(B, SKV, 16 * DH),
        V_ext.reshape(B, SKV, 16 * DH),
        Wo,
    )


# device time: 13610 ns/iter; 1.1994x vs baseline; 1.1289x over previous
import jax
import jax.numpy as jnp
from jax import lax
from jax.experimental import pallas as pl
from jax.experimental.pallas import tpu as pltpu

N_DEV = 4
B, SQ, SKV, DH = 2, 128, 128, 64
H_LOC = 4
D_MODEL = 512
D_LOC = H_LOC * DH
N_COL = 2
D_CHUNK = D_MODEL // N_COL
N_CHUNK = B * N_COL


def kernel(x, Wq, K_ext, V_ext, Wo):
    def body(x_ref, wq_ref, k_ref, v_ref, wo_ref, out_ref,
             send_ref, recv_ref, send_sems, recv_sems):
        my_pos = lax.axis_index("i")
        pa = my_pos ^ 1
        pb = (N_DEV - 1) - my_pos

        def partners(c):
            return (pa, pb) if c % 2 == 0 else (pb, pa)

        barrier_sem = pltpu.get_barrier_semaphore()
        for nbr in (pa, pb):
            pl.semaphore_signal(
                barrier_sem, inc=1,
                device_id=(nbr,), device_id_type=pl.DeviceIdType.MESH,
            )

        wq = wq_ref[:].astype(jnp.bfloat16)
        wo = wo_ref[:].astype(jnp.bfloat16)

        def exchange(round_idx, c, partner):
            return pltpu.make_async_remote_copy(
                src_ref=send_ref.at[round_idx, c],
                dst_ref=recv_ref.at[round_idx, c],
                send_sem=send_sems.at[round_idx, c],
                recv_sem=recv_sems.at[round_idx, c],
                device_id=(partner,),
                device_id_type=pl.DeviceIdType.MESH,
            )

        r1 = []
        for b in range(B):
            xb = x_ref[b].astype(jnp.bfloat16)
            q_b = lax.dot(xb, wq, preferred_element_type=jnp.float32)
            q_b = (q_b * 0.125).astype(jnp.bfloat16)
            ctx = []
            for h in range(H_LOC):
                q_bh = q_b[:, h * DH:(h + 1) * DH]
                k_bh = k_ref[b, :, h * DH:(h + 1) * DH].astype(jnp.bfloat16)
                v_bh = v_ref[b, :, h * DH:(h + 1) * DH].astype(jnp.bfloat16)
                s = lax.dot_general(
                    q_bh, k_bh, (((1,), (1,)), ((), ())),
                    preferred_element_type=jnp.float32,
                )
                e = jnp.exp(s)
                recip = 1.0 / jnp.sum(e, axis=1, keepdims=True)
                ctx_bh = lax.dot(
                    e.astype(jnp.bfloat16), v_bh,
                    preferred_element_type=jnp.float32,
                ) * recip
                ctx.append(ctx_bh.astype(jnp.bfloat16))

            ctx_b = jnp.concatenate(ctx, axis=1)
            for j in range(N_COL):
                c = b * N_COL + j
                cols = slice(j * D_CHUNK, (j + 1) * D_CHUNK)
                partial = lax.dot(
                    ctx_b, wo[:, cols], preferred_element_type=jnp.float32
                )
                out_ref[b, :, cols] = partial
                send_ref[0, c] = partial.astype(jnp.bfloat16)
                if c == 0:
                    pl.semaphore_wait(barrier_sem, 2)
                rdma = exchange(0, c, partners(c)[0])
                rdma.start()
                r1.append(rdma)

        r2 = []
        for c in range(N_CHUNK):
            b, j = divmod(c, N_COL)
            cols = slice(j * D_CHUNK, (j + 1) * D_CHUNK)
            r1[c].wait_recv()
            out_ref[b, :, cols] += recv_ref[0, c].astype(jnp.float32)
            send_ref[1, c] = out_ref[b, :, cols].astype(jnp.bfloat16)
            rdma = exchange(1, c, partners(c)[1])
            rdma.start()
            r2.append(rdma)

        for c in range(N_CHUNK):
            b, j = divmod(c, N_COL)
            cols = slice(j * D_CHUNK, (j + 1) * D_CHUNK)
            r2[c].wait_recv()
            out_ref[b, :, cols] += recv_ref[1, c].astype(jnp.float32)

        for rdma in r1 + r2:
            rdma.wait_send()

    def kv_map(i):
        return (0, 0, lax.axis_index("i"))

    return pl.pallas_call(
        body,
        grid=(1,),
        out_shape=jax.ShapeDtypeStruct((B, SQ, D_MODEL), jnp.float32),
        in_specs=[
            pl.BlockSpec((B, SQ, D_MODEL), lambda i: (0, 0, 0)),
            pl.BlockSpec((D_MODEL, D_LOC), lambda i: (0, 0)),
            pl.BlockSpec((B, SKV, D_LOC), kv_map),
            pl.BlockSpec((B, SKV, D_LOC), kv_map),
            pl.BlockSpec((D_LOC, D_MODEL), lambda i: (0, 0)),
        ],
        out_specs=pl.BlockSpec((B, SQ, D_MODEL), lambda i: (0, 0, 0)),
        scratch_shapes=[
            pltpu.VMEM((2, N_CHUNK, SQ, D_CHUNK), jnp.bfloat16),
            pltpu.VMEM((2, N_CHUNK, SQ, D_CHUNK), jnp.bfloat16),
            pltpu.SemaphoreType.DMA((2, N_CHUNK)),
            pltpu.SemaphoreType.DMA((2, N_CHUNK)),
        ],
        compiler_params=pltpu.CompilerParams(collective_id=0),
    )(
        x,
        Wq,
        K_ext.reshape(B, SKV, 16 * DH),
        V_ext.reshape(B, SKV, 16 * DH),
        Wo,
    )


# device time: 13607 ns/iter; 1.1997x vs baseline; 1.0002x over previous
import jax
import jax.numpy as jnp
from jax import lax
from jax.experimental import pallas as pl
from jax.experimental.pallas import tpu as pltpu

N_DEV = 4
B, SQ, SKV, DH = 2, 128, 128, 64
H_LOC = 4
D_MODEL = 512
D_LOC = H_LOC * DH
N_COL = 2
D_CHUNK = D_MODEL // N_COL
N_CHUNK = B * N_COL


def kernel(x, Wq, K_ext, V_ext, Wo):
    def body(x_ref, wq_ref, k_ref, v_ref, wo_ref, out_ref,
             send_ref, recv_ref, send_sems, recv_sems):
        my_pos = lax.axis_index("i")
        pa = my_pos ^ 1
        pb = (N_DEV - 1) - my_pos

        def partners(c):
            return (pa, pb) if c % 2 == 0 else (pb, pa)

        barrier_sem = pltpu.get_barrier_semaphore()
        for nbr in (pa, pb):
            pl.semaphore_signal(
                barrier_sem, inc=1,
                device_id=(nbr,), device_id_type=pl.DeviceIdType.MESH,
            )

        wq = wq_ref[:].astype(jnp.bfloat16)
        wo = wo_ref[:].astype(jnp.bfloat16)

        def exchange(round_idx, c, partner):
            return pltpu.make_async_remote_copy(
                src_ref=send_ref.at[round_idx, c],
                dst_ref=recv_ref.at[round_idx, c],
                send_sem=send_sems.at[round_idx, c],
                recv_sem=recv_sems.at[round_idx, c],
                device_id=(partner,),
                device_id_type=pl.DeviceIdType.MESH,
            )

        r1 = []
        for b in range(B):
            xb = x_ref[b].astype(jnp.bfloat16)
            q_b = lax.dot(xb, wq, preferred_element_type=jnp.float32)
            q_b = (q_b * 0.125).astype(jnp.bfloat16)
            ctx = []
            for h in range(H_LOC):
                q_bh = q_b[:, h * DH:(h + 1) * DH]
                k_bh = k_ref[b, :, h * DH:(h + 1) * DH].astype(jnp.bfloat16)
                v_bh = v_ref[b, :, h * DH:(h + 1) * DH].astype(jnp.bfloat16)
                s = lax.dot_general(
                    q_bh, k_bh, (((1,), (1,)), ((), ())),
                    preferred_element_type=jnp.float32,
                )
                e = jnp.exp(s)
                recip = 1.0 / jnp.sum(e, axis=1, keepdims=True)
                ctx_bh = lax.dot(
                    e.astype(jnp.bfloat16), v_bh,
                    preferred_element_type=jnp.float32,
                ) * recip
                ctx.append(ctx_bh.astype(jnp.bfloat16))

            ctx_b = jnp.concatenate(ctx, axis=1)
            for j in range(N_COL):
                c = b * N_COL + j
                cols = slice(j * D_CHUNK, (j + 1) * D_CHUNK)
                partial = lax.dot(
                    ctx_b, wo[:, cols], preferred_element_type=jnp.float32
                )
                send_ref[0, c] = partial.astype(jnp.bfloat16)
                if c == 0:
                    pl.semaphore_wait(barrier_sem, 2)
                rdma = exchange(0, c, partners(c)[0])
                rdma.start()
                r1.append(rdma)

        r2 = []
        for c in range(N_CHUNK):
            r1[c].wait_recv()
            send_ref[1, c] = send_ref[0, c] + recv_ref[0, c]
            rdma = exchange(1, c, partners(c)[1])
            rdma.start()
            r2.append(rdma)

        for c in range(N_CHUNK):
            b, j = divmod(c, N_COL)
            cols = slice(j * D_CHUNK, (j + 1) * D_CHUNK)
            r2[c].wait_recv()
            out_ref[b, :, cols] = (
                send_ref[1, c].astype(jnp.float32)
                + recv_ref[1, c].astype(jnp.float32)
            )

        for rdma in r1 + r2:
            rdma.wait_send()

    def kv_map(i):
        return (0, 0, lax.axis_index("i"))

    return pl.pallas_call(
        body,
        grid=(1,),
        out_shape=jax.ShapeDtypeStruct((B, SQ, D_MODEL), jnp.float32),
        in_specs=[
            pl.BlockSpec((B, SQ, D_MODEL), lambda i: (0, 0, 0)),
            pl.BlockSpec((D_MODEL, D_LOC), lambda i: (0, 0)),
            pl.BlockSpec((B, SKV, D_LOC), kv_map),
            pl.BlockSpec((B, SKV, D_LOC), kv_map),
            pl.BlockSpec((D_LOC, D_MODEL), lambda i: (0, 0)),
        ],
        out_specs=pl.BlockSpec((B, SQ, D_MODEL), lambda i: (0, 0, 0)),
        scratch_shapes=[
            pltpu.VMEM((2, N_CHUNK, SQ, D_CHUNK), jnp.bfloat16),
            pltpu.VMEM((2, N_CHUNK, SQ, D_CHUNK), jnp.bfloat16),
            pltpu.SemaphoreType.DMA((2, N_CHUNK)),
            pltpu.SemaphoreType.DMA((2, N_CHUNK)),
        ],
        compiler_params=pltpu.CompilerParams(collective_id=0),
    )(
        x,
        Wq,
        K_ext.reshape(B, SKV, 16 * DH),
        V_ext.reshape(B, SKV, 16 * DH),
        Wo,
    )


# device time: 13405 ns/iter; 1.2178x vs baseline; 1.0151x over previous
import jax
import jax.numpy as jnp
from jax import lax
from jax.experimental import pallas as pl
from jax.experimental.pallas import tpu as pltpu

N_DEV = 4
B, SQ, SKV, DH = 2, 128, 128, 64
H_LOC = 4
D_MODEL = 512
D_LOC = H_LOC * DH
N_COL = 4
D_CHUNK = D_MODEL // N_COL
N_CHUNK = B * N_COL


def kernel(x, Wq, K_ext, V_ext, Wo):
    def body(x_ref, wq_ref, k_ref, v_ref, wo_ref, out_ref,
             send_ref, recv_ref, send_sems, recv_sems):
        my_pos = lax.axis_index("i")
        pa = my_pos ^ 1
        pb = (N_DEV - 1) - my_pos

        def partners(c):
            return (pa, pb) if c % 2 == 0 else (pb, pa)

        barrier_sem = pltpu.get_barrier_semaphore()
        for nbr in (pa, pb):
            pl.semaphore_signal(
                barrier_sem, inc=1,
                device_id=(nbr,), device_id_type=pl.DeviceIdType.MESH,
            )

        wq = wq_ref[:].astype(jnp.bfloat16)
        wo = wo_ref[:].astype(jnp.bfloat16)

        def exchange(round_idx, c, partner):
            return pltpu.make_async_remote_copy(
                src_ref=send_ref.at[round_idx, c],
                dst_ref=recv_ref.at[round_idx, c],
                send_sem=send_sems.at[round_idx, c],
                recv_sem=recv_sems.at[round_idx, c],
                device_id=(partner,),
                device_id_type=pl.DeviceIdType.MESH,
            )

        r1 = []
        for b in range(B):
            xb = x_ref[b].astype(jnp.bfloat16)
            q_b = lax.dot(xb, wq, preferred_element_type=jnp.float32)
            q_b = (q_b * 0.125).astype(jnp.bfloat16)
            ctx = []
            for h in range(H_LOC):
                q_bh = q_b[:, h * DH:(h + 1) * DH]
                k_bh = k_ref[b, :, h * DH:(h + 1) * DH].astype(jnp.bfloat16)
                v_bh = v_ref[b, :, h * DH:(h + 1) * DH].astype(jnp.bfloat16)
                s = lax.dot_general(
                    q_bh, k_bh, (((1,), (1,)), ((), ())),
                    preferred_element_type=jnp.float32,
                )
                e = jnp.exp(s)
                recip = 1.0 / jnp.sum(e, axis=1, keepdims=True)
                ctx_bh = lax.dot(
                    e.astype(jnp.bfloat16), v_bh,
                    preferred_element_type=jnp.float32,
                ) * recip
                ctx.append(ctx_bh.astype(jnp.bfloat16))

            ctx_b = jnp.concatenate(ctx, axis=1)
            for j in range(N_COL):
                c = b * N_COL + j
                cols = slice(j * D_CHUNK, (j + 1) * D_CHUNK)
                partial = lax.dot(
                    ctx_b, wo[:, cols], preferred_element_type=jnp.float32
                )
                send_ref[0, c] = partial.astype(jnp.bfloat16)
                if c == 0:
                    pl.semaphore_wait(barrier_sem, 2)
                rdma = exchange(0, c, partners(c)[0])
                rdma.start()
                r1.append(rdma)

        r2 = []
        for c in range(N_CHUNK):
            r1[c].wait_recv()
            send_ref[1, c] = send_ref[0, c] + recv_ref[0, c]
            rdma = exchange(1, c, partners(c)[1])
            rdma.start()
            r2.append(rdma)

        for c in range(N_CHUNK):
            b, j = divmod(c, N_COL)
            cols = slice(j * D_CHUNK, (j + 1) * D_CHUNK)
            r2[c].wait_recv()
            out_ref[b, :, cols] = (
                send_ref[1, c].astype(jnp.float32)
                + recv_ref[1, c].astype(jnp.float32)
            )

        for rdma in r1 + r2:
            rdma.wait_send()

    def kv_map(i):
        return (0, 0, lax.axis_index("i"))

    return pl.pallas_call(
        body,
        grid=(1,),
        out_shape=jax.ShapeDtypeStruct((B, SQ, D_MODEL), jnp.float32),
        in_specs=[
            pl.BlockSpec((B, SQ, D_MODEL), lambda i: (0, 0, 0)),
            pl.BlockSpec((D_MODEL, D_LOC), lambda i: (0, 0)),
            pl.BlockSpec((B, SKV, D_LOC), kv_map),
            pl.BlockSpec((B, SKV, D_LOC), kv_map),
            pl.BlockSpec((D_LOC, D_MODEL), lambda i: (0, 0)),
        ],
        out_specs=pl.BlockSpec((B, SQ, D_MODEL), lambda i: (0, 0, 0)),
        scratch_shapes=[
            pltpu.VMEM((2, N_CHUNK, SQ, D_CHUNK), jnp.bfloat16),
            pltpu.VMEM((2, N_CHUNK, SQ, D_CHUNK), jnp.bfloat16),
            pltpu.SemaphoreType.DMA((2, N_CHUNK)),
            pltpu.SemaphoreType.DMA((2, N_CHUNK)),
        ],
        compiler_params=pltpu.CompilerParams(collective_id=0),
    )(
        x,
        Wq,
        K_ext.reshape(B, SKV, 16 * DH),
        V_ext.reshape(B, SKV, 16 * DH),
        Wo,
    )
